# baseline (speedup 1.0000x reference)
import functools

import jax
import jax.numpy as jnp
from jax import lax
from jax.experimental import pallas as pl
from jax.experimental.pallas import tpu as pltpu
from jax.experimental.pallas import tpu_sc as plsc

IMG_SIZE = 600.0
ALPHA = 0.25
GAMMA = 2.0

_NC = 2
_NS = 16
_LANES = 16
_NW = _NC * _NS

_BA = 34816


def _match_body(T, CH, steps, boxs_hbm, boxg_hbm, anch_hbm, out_hbm,
                boxsv, boxgv, anchv, outv):
    wid = lax.axis_index("c") * _NS + lax.axis_index("s")
    base = wid * CH
    pltpu.sync_copy(boxs_hbm, boxsv)
    pltpu.sync_copy(boxg_hbm, boxgv)
    pltpu.sync_copy(anch_hbm.at[:, pl.ds(base, CH)], anchv)

    UNROLL = 2

    def step(i, carry):
        sls, ax1s, ay1s, ax2ps, ay2ps, a1s = [], [], [], [], [], []
        bnums, bdens, bidxs = [], [], []
        for u in range(UNROLL):
            sl = pl.ds((i * UNROLL + u) * _LANES, _LANES)
            sls.append(sl)
            ax = anchv[0, sl]
            ay = anchv[1, sl]
            aw = anchv[2, sl]
            ah = anchv[3, sl]
            ax1 = ax - aw * 0.5
            ay1 = ay - ah * 0.5
            ax2 = ax + aw * 0.5
            ay2 = ay + ah * 0.5
            ax1s.append(ax1)
            ay1s.append(ay1)
            ax2ps.append(ax2 + 1.0)
            ay2ps.append(ay2 + 1.0)
            a1s.append(((ax2 - ax1) + 1.0) * ((ay2 - ay1) + 1.0))
            bnums.append(jnp.zeros((_LANES,), jnp.float32))
            bdens.append(jnp.ones((_LANES,), jnp.float32))
            bidxs.append(jnp.zeros((_LANES,), jnp.int32))
        for t in range(T):
            x1t = boxsv[0, t]
            y1t = boxsv[1, t]
            x2pt = boxsv[2, t]
            y2pt = boxsv[3, t]
            at = boxsv[4, t]
            for u in range(UNROLL):
                ltx = jnp.maximum(ax1s[u], x1t)
                lty = jnp.maximum(ay1s[u], y1t)
                rxp = jnp.minimum(ax2ps[u], x2pt)
                ryp = jnp.minimum(ay2ps[u], y2pt)
                wx = jnp.maximum(rxp - ltx, 0.0)
                wy = jnp.maximum(ryp - lty, 0.0)
                inter = wx * wy
                s = a1s[u] + at
                better = inter * bdens[u] > bnums[u] * s
                bnums[u] = jnp.where(better, inter, bnums[u])
                bdens[u] = jnp.where(better, s, bdens[u])
                bidxs[u] = jnp.where(better, t, bidxs[u])
        for u in range(UNROLL):
            sl = sls[u]
            bidx = bidxs[u]
            miou = bnums[u] / (bdens[u] - bnums[u])
            lab = plsc.load_gather(boxgv, [bidx])
            bcx = plsc.load_gather(boxgv, [bidx + T])
            bcy = plsc.load_gather(boxgv, [bidx + 2 * T])
            bw = plsc.load_gather(boxgv, [bidx + 3 * T])
            bh = plsc.load_gather(boxgv, [bidx + 4 * T])
            ax = anchv[0, sl]
            ay = anchv[1, sl]
            aw = anchv[2, sl]
            ah = anchv[3, sl]
            ct = lab + 1.0
            ct = jnp.where(miou < 0.5, 0.0, ct)
            ct = jnp.where((miou > 0.4) & (miou < 0.5), -1.0, ct)
            outv[0, sl] = ct
            outv[1, sl] = (bcx - ax) / aw
            outv[2, sl] = (bcy - ay) / ah
            outv[3, sl] = bw / aw
            outv[4, sl] = bh / ah
        return carry

    lax.fori_loop(0, steps // UNROLL, step, 0)
    pltpu.sync_copy(outv, out_hbm.at[:, pl.ds(base, CH)])


def _fold_lanes(v, width):
    while width > 128 and width % 256 == 0:
        width //= 2
        v = v[:, :width] + v[:, width:]
    if width > 128:
        acc = v[:, 0:128]
        for k in range(1, width // 128):
            acc = acc + v[:, 128 * k:128 * (k + 1)]
        v = acc
    return v


def _f01(x):
    e = jnp.exp(-jnp.abs(x))
    s = jnp.maximum(x, 0.0) + jnp.log1p(e)
    t = x - s
    p2 = jnp.exp(t + t)
    q2 = jnp.exp(-(s + s))
    f0 = (1.0 - ALPHA) * p2 * s
    f1 = ALPHA * q2 * (-t)
    return f0, f1


def _fold20(f, C):
    t8 = f[0:8, :] + f[8:16, :]
    v4 = (t8[0:4, :] + t8[4:8, :]) + f[16:C, :]
    v2 = v4[0:2, :] + v4[2:4, :]
    return v2[0:1, :] + v2[1:2, :]


def _bg_body(C, A, BA, cls_ref, out_ref):
    lane = lax.broadcasted_iota(jnp.int32, (1, BA), 1)
    valid = (pl.program_id(0) * BA + lane) < A
    x = cls_ref[...]
    f0, _ = _f01(x)
    f0 = jnp.where(valid, f0, 0.0)
    out_ref[...] = _fold20(f0, C)


def _loss_body(C, A, BA, nsteps, cls_ref, loc_ref, match_ref, s0_ref,
               out_ref, acc):
    @pl.when(pl.program_id(0) == 0)
    def _init():
        acc[...] = jnp.zeros_like(acc)

    ct = match_ref[0:1, :]
    pos = ct > 0.0
    posf = pos.astype(jnp.float32)
    pn = (ct > -0.5).astype(jnp.float32)

    def _sl1(lp, enc):
        d = lp - jnp.where(pos, enc, 0.0)
        ad = jnp.abs(d)
        return jnp.where(ad < 1.0, 0.5 * d * d, ad - 0.5)

    sl1 = _sl1(loc_ref[0:2, :], match_ref[1:3, :]) + _sl1(
        loc_ref[2:4, :], jnp.log(match_ref[3:5, :]))
    loc_v = jnp.where(pos, sl1, 0.0)

    x = cls_ref[...]
    cti = ct.astype(jnp.int32) - 1
    cls_id = lax.broadcasted_iota(jnp.int32, x.shape, 0)
    xsel = _fold20(jnp.where(cls_id == cti, x, 0.0), C)
    g0, g1 = _f01(xsel)
    cls_v = pn * s0_ref[...] + jnp.where(pos, g1 - g0, 0.0)

    acc[0:1, :] += _fold_lanes(cls_v, cls_v.shape[1])
    acc[4:6, :] += _fold_lanes(loc_v, loc_v.shape[1])
    acc[8:9, :] += _fold_lanes(posf, posf.shape[1])

    @pl.when(pl.program_id(0) == nsteps - 1)
    def _fin():
        loc_s = jnp.sum(acc[4:6, :])
        cls_s = jnp.sum(acc[0:1, :])
        np_s = jnp.maximum(1.0, jnp.sum(acc[8:9, :]))
        out_ref[0] = (loc_s + cls_s) / np_s
        out_ref[1] = loc_s / np_s
        out_ref[2] = cls_s / np_s


def kernel(loc_preds, cls_preds, targets, iou_boxes):
    A = iou_boxes.shape[0]
    T = targets.shape[0]
    C = cls_preds.shape[-1]
    grid_n = -(-A // _BA)
    a_pad = -(-grid_n * _BA // (_NW * _LANES)) * _NW * _LANES
    grid_n = a_pad // _BA
    CH = a_pad // _NW
    steps = CH // _LANES

    boxes = targets[:, 2:] * IMG_SIZE
    labels = targets[:, 1]
    half = boxes[:, 2:] * 0.5
    xy1 = boxes[:, :2] - half
    xy2 = boxes[:, :2] + half
    area = ((xy2[:, 0] - xy1[:, 0]) + 1.0) * ((xy2[:, 1] - xy1[:, 1]) + 1.0)
    box_scan = jnp.stack(
        [xy1[:, 0], xy1[:, 1], xy2[:, 0] + 1.0, xy2[:, 1] + 1.0, area], axis=0)
    box_scan = jnp.broadcast_to(box_scan[:, :, None], (5, T, _LANES))
    box_gath = jnp.concatenate(
        [labels, boxes[:, 0], boxes[:, 1], boxes[:, 2], boxes[:, 3]], axis=0)

    npad = a_pad - A
    anch_pad = jnp.tile(
        jnp.array([[-1e6], [-1e6], [1.0], [1.0]], jnp.float32), (1, npad))
    anchT = jnp.concatenate([iou_boxes.T, anch_pad], axis=1)
    clsT = cls_preds[0].T
    locT = loc_preds[0].T

    mesh = plsc.VectorSubcoreMesh(
        core_axis_name="c", subcore_axis_name="s",
        num_cores=_NC, num_subcores=_NS)
    match = pl.kernel(
        functools.partial(_match_body, T, CH, steps),
        out_type=jax.ShapeDtypeStruct((5, a_pad), jnp.float32),
        mesh=mesh,
        scratch_types=[
            pltpu.VMEM((5, T, _LANES), jnp.float32),
            pltpu.VMEM((5 * T,), jnp.float32),
            pltpu.VMEM((4, CH), jnp.float32),
            pltpu.VMEM((5, CH), jnp.float32),
        ],
        compiler_params=pltpu.CompilerParams(needs_layout_passes=False),
    )(box_scan, box_gath, anchT)

    s0 = pl.pallas_call(
        functools.partial(_bg_body, C, A, _BA),
        grid=(grid_n,),
        in_specs=[pl.BlockSpec((C, _BA), lambda i: (0, i))],
        out_specs=pl.BlockSpec((1, _BA), lambda i: (0, i)),
        out_shape=jax.ShapeDtypeStruct((1, a_pad), jnp.float32),
    )(clsT)

    sums = pl.pallas_call(
        functools.partial(_loss_body, C, A, _BA, grid_n),
        grid=(grid_n,),
        in_specs=[
            pl.BlockSpec((C, _BA), lambda i: (0, i)),
            pl.BlockSpec((4, _BA), lambda i: (0, i)),
            pl.BlockSpec((5, _BA), lambda i: (0, i)),
            pl.BlockSpec((1, _BA), lambda i: (0, i)),
        ],
        out_specs=pl.BlockSpec(memory_space=pltpu.SMEM),
        out_shape=jax.ShapeDtypeStruct((3,), jnp.float32),
        scratch_shapes=[pltpu.VMEM((16, 128), jnp.float32)],
    )(clsT, locT, match, s0)

    return (sums[0], sums[1], sums[2])


# speedup vs baseline: 1.0213x; 1.0213x over previous
import functools

import jax
import jax.numpy as jnp
from jax import lax
from jax.experimental import pallas as pl
from jax.experimental.pallas import tpu as pltpu
from jax.experimental.pallas import tpu_sc as plsc

IMG_SIZE = 600.0
ALPHA = 0.25
GAMMA = 2.0

_NC = 2
_NS = 16
_LANES = 16
_NW = _NC * _NS

_BA = 8704


def _match_body(T, CH, steps, boxs_hbm, boxg_hbm, anch_hbm, out_hbm,
                boxsv, boxgv, anchv, outv):
    wid = lax.axis_index("c") * _NS + lax.axis_index("s")
    base = wid * CH
    pltpu.sync_copy(boxs_hbm, boxsv)
    pltpu.sync_copy(boxg_hbm, boxgv)
    pltpu.sync_copy(anch_hbm.at[:, pl.ds(base, CH)], anchv)

    UNROLL = 2

    def step(i, carry):
        sls, ax1s, ay1s, ax2ps, ay2ps, a1s = [], [], [], [], [], []
        bnums, bdens, bidxs = [], [], []
        for u in range(UNROLL):
            sl = pl.ds((i * UNROLL + u) * _LANES, _LANES)
            sls.append(sl)
            ax = anchv[0, sl]
            ay = anchv[1, sl]
            aw = anchv[2, sl]
            ah = anchv[3, sl]
            ax1 = ax - aw * 0.5
            ay1 = ay - ah * 0.5
            ax2 = ax + aw * 0.5
            ay2 = ay + ah * 0.5
            ax1s.append(ax1)
            ay1s.append(ay1)
            ax2ps.append(ax2 + 1.0)
            ay2ps.append(ay2 + 1.0)
            a1s.append(((ax2 - ax1) + 1.0) * ((ay2 - ay1) + 1.0))
            bnums.append(jnp.zeros((_LANES,), jnp.float32))
            bdens.append(jnp.ones((_LANES,), jnp.float32))
            bidxs.append(jnp.zeros((_LANES,), jnp.int32))
        for t in range(T):
            x1t = boxsv[0, t]
            y1t = boxsv[1, t]
            x2pt = boxsv[2, t]
            y2pt = boxsv[3, t]
            at = boxsv[4, t]
            for u in range(UNROLL):
                ltx = jnp.maximum(ax1s[u], x1t)
                lty = jnp.maximum(ay1s[u], y1t)
                rxp = jnp.minimum(ax2ps[u], x2pt)
                ryp = jnp.minimum(ay2ps[u], y2pt)
                wx = jnp.maximum(rxp - ltx, 0.0)
                wy = jnp.maximum(ryp - lty, 0.0)
                inter = wx * wy
                s = a1s[u] + at
                better = inter * bdens[u] > bnums[u] * s
                bnums[u] = jnp.where(better, inter, bnums[u])
                bdens[u] = jnp.where(better, s, bdens[u])
                bidxs[u] = jnp.where(better, t, bidxs[u])
        for u in range(UNROLL):
            sl = sls[u]
            bidx = bidxs[u]
            miou = bnums[u] / (bdens[u] - bnums[u])
            lab = plsc.load_gather(boxgv, [bidx])
            bcx = plsc.load_gather(boxgv, [bidx + T])
            bcy = plsc.load_gather(boxgv, [bidx + 2 * T])
            bw = plsc.load_gather(boxgv, [bidx + 3 * T])
            bh = plsc.load_gather(boxgv, [bidx + 4 * T])
            ax = anchv[0, sl]
            ay = anchv[1, sl]
            aw = anchv[2, sl]
            ah = anchv[3, sl]
            ct = lab + 1.0
            ct = jnp.where(miou < 0.5, 0.0, ct)
            ct = jnp.where((miou > 0.4) & (miou < 0.5), -1.0, ct)
            outv[0, sl] = ct
            outv[1, sl] = (bcx - ax) / aw
            outv[2, sl] = (bcy - ay) / ah
            outv[3, sl] = bw / aw
            outv[4, sl] = bh / ah
        return carry

    lax.fori_loop(0, steps // UNROLL, step, 0)
    pltpu.sync_copy(outv, out_hbm.at[:, pl.ds(base, CH)])


def _fold_lanes(v, width):
    while width > 128 and width % 256 == 0:
        width //= 2
        v = v[:, :width] + v[:, width:]
    if width > 128:
        acc = v[:, 0:128]
        for k in range(1, width // 128):
            acc = acc + v[:, 128 * k:128 * (k + 1)]
        v = acc
    return v


def _f01(x):
    e = jnp.exp(-jnp.abs(x))
    s = jnp.maximum(x, 0.0) + jnp.log1p(e)
    t = x - s
    p2 = jnp.exp(t + t)
    q2 = jnp.exp(-(s + s))
    f0 = (1.0 - ALPHA) * p2 * s
    f1 = ALPHA * q2 * (-t)
    return f0, f1


def _fold20(f, C):
    t8 = f[0:8, :] + f[8:16, :]
    v4 = (t8[0:4, :] + t8[4:8, :]) + f[16:C, :]
    v2 = v4[0:2, :] + v4[2:4, :]
    return v2[0:1, :] + v2[1:2, :]


def _bg_body(C, A, BA, cls_ref, out_ref):
    lane = lax.broadcasted_iota(jnp.int32, (1, BA), 1)
    valid = (pl.program_id(0) * BA + lane) < A
    x = cls_ref[...]
    f0, _ = _f01(x)
    f0 = jnp.where(valid, f0, 0.0)
    out_ref[...] = _fold20(f0, C)


def _loss_body(C, A, BA, nsteps, cls_ref, loc_ref, match_ref, s0_ref,
               out_ref, acc):
    @pl.when(pl.program_id(0) == 0)
    def _init():
        acc[...] = jnp.zeros_like(acc)

    ct = match_ref[0:1, :]
    pos = ct > 0.0
    posf = pos.astype(jnp.float32)
    pn = (ct > -0.5).astype(jnp.float32)

    def _sl1(lp, enc):
        d = lp - jnp.where(pos, enc, 0.0)
        ad = jnp.abs(d)
        return jnp.where(ad < 1.0, 0.5 * d * d, ad - 0.5)

    sl1 = _sl1(loc_ref[0:2, :], match_ref[1:3, :]) + _sl1(
        loc_ref[2:4, :], jnp.log(match_ref[3:5, :]))
    loc_v = jnp.where(pos, sl1, 0.0)

    x = cls_ref[...]
    cti = ct.astype(jnp.int32) - 1
    cls_id = lax.broadcasted_iota(jnp.int32, x.shape, 0)
    xsel = _fold20(jnp.where(cls_id == cti, x, 0.0), C)
    g0, g1 = _f01(xsel)
    cls_v = pn * s0_ref[...] + jnp.where(pos, g1 - g0, 0.0)

    acc[0:1, :] += _fold_lanes(cls_v, cls_v.shape[1])
    acc[4:6, :] += _fold_lanes(loc_v, loc_v.shape[1])
    acc[8:9, :] += _fold_lanes(posf, posf.shape[1])

    @pl.when(pl.program_id(0) == nsteps - 1)
    def _fin():
        loc_s = jnp.sum(acc[4:6, :])
        cls_s = jnp.sum(acc[0:1, :])
        np_s = jnp.maximum(1.0, jnp.sum(acc[8:9, :]))
        out_ref[0] = (loc_s + cls_s) / np_s
        out_ref[1] = loc_s / np_s
        out_ref[2] = cls_s / np_s


def kernel(loc_preds, cls_preds, targets, iou_boxes):
    A = iou_boxes.shape[0]
    T = targets.shape[0]
    C = cls_preds.shape[-1]
    grid_n = -(-A // _BA)
    a_pad = -(-grid_n * _BA // (_NW * _LANES)) * _NW * _LANES
    grid_n = a_pad // _BA
    CH = a_pad // _NW
    steps = CH // _LANES

    boxes = targets[:, 2:] * IMG_SIZE
    labels = targets[:, 1]
    half = boxes[:, 2:] * 0.5
    xy1 = boxes[:, :2] - half
    xy2 = boxes[:, :2] + half
    area = ((xy2[:, 0] - xy1[:, 0]) + 1.0) * ((xy2[:, 1] - xy1[:, 1]) + 1.0)
    box_scan = jnp.stack(
        [xy1[:, 0], xy1[:, 1], xy2[:, 0] + 1.0, xy2[:, 1] + 1.0, area], axis=0)
    box_scan = jnp.broadcast_to(box_scan[:, :, None], (5, T, _LANES))
    box_gath = jnp.concatenate(
        [labels, boxes[:, 0], boxes[:, 1], boxes[:, 2], boxes[:, 3]], axis=0)

    npad = a_pad - A
    anch_pad = jnp.tile(
        jnp.array([[-1e6], [-1e6], [1.0], [1.0]], jnp.float32), (1, npad))
    anchT = jnp.concatenate([iou_boxes.T, anch_pad], axis=1)
    clsT = cls_preds[0].T
    locT = loc_preds[0].T

    mesh = plsc.VectorSubcoreMesh(
        core_axis_name="c", subcore_axis_name="s",
        num_cores=_NC, num_subcores=_NS)
    match = pl.kernel(
        functools.partial(_match_body, T, CH, steps),
        out_type=jax.ShapeDtypeStruct((5, a_pad), jnp.float32),
        mesh=mesh,
        scratch_types=[
            pltpu.VMEM((5, T, _LANES), jnp.float32),
            pltpu.VMEM((5 * T,), jnp.float32),
            pltpu.VMEM((4, CH), jnp.float32),
            pltpu.VMEM((5, CH), jnp.float32),
        ],
        compiler_params=pltpu.CompilerParams(needs_layout_passes=False),
    )(box_scan, box_gath, anchT)

    s0 = pl.pallas_call(
        functools.partial(_bg_body, C, A, _BA),
        grid=(grid_n,),
        in_specs=[pl.BlockSpec((C, _BA), lambda i: (0, i))],
        out_specs=pl.BlockSpec((1, _BA), lambda i: (0, i)),
        out_shape=jax.ShapeDtypeStruct((1, a_pad), jnp.float32),
    )(clsT)

    sums = pl.pallas_call(
        functools.partial(_loss_body, C, A, _BA, grid_n),
        grid=(grid_n,),
        in_specs=[
            pl.BlockSpec((C, _BA), lambda i: (0, i)),
            pl.BlockSpec((4, _BA), lambda i: (0, i)),
            pl.BlockSpec((5, _BA), lambda i: (0, i)),
            pl.BlockSpec((1, _BA), lambda i: (0, i)),
        ],
        out_specs=pl.BlockSpec(memory_space=pltpu.SMEM),
        out_shape=jax.ShapeDtypeStruct((3,), jnp.float32),
        scratch_shapes=[pltpu.VMEM((16, 128), jnp.float32)],
    )(clsT, locT, match, s0)

    return (sums[0], sums[1], sums[2])


# speedup vs baseline: 1.0270x; 1.0056x over previous
import functools

import jax
import jax.numpy as jnp
from jax import lax
from jax.experimental import pallas as pl
from jax.experimental.pallas import tpu as pltpu
from jax.experimental.pallas import tpu_sc as plsc

IMG_SIZE = 600.0
ALPHA = 0.25
GAMMA = 2.0

_NC = 2
_NS = 16
_LANES = 16
_NW = _NC * _NS

_BA = 8704


def _match_body(T, CH, steps, boxs_hbm, boxg_hbm, anch_hbm, out_hbm,
                boxsv, boxgv, anchv, outv):
    wid = lax.axis_index("c") * _NS + lax.axis_index("s")
    base = wid * CH
    pltpu.sync_copy(boxs_hbm, boxsv)
    pltpu.sync_copy(boxg_hbm, boxgv)
    pltpu.sync_copy(anch_hbm.at[:, pl.ds(base, CH)], anchv)

    UNROLL = 3

    def step(i, carry):
        sls, ax1s, ay1s, ax2ps, ay2ps, a1s = [], [], [], [], [], []
        bnums, bdens, bidxs = [], [], []
        for u in range(UNROLL):
            sl = pl.ds((i * UNROLL + u) * _LANES, _LANES)
            sls.append(sl)
            ax = anchv[0, sl]
            ay = anchv[1, sl]
            aw = anchv[2, sl]
            ah = anchv[3, sl]
            ax1 = ax - aw * 0.5
            ay1 = ay - ah * 0.5
            ax2 = ax + aw * 0.5
            ay2 = ay + ah * 0.5
            ax1s.append(ax1)
            ay1s.append(ay1)
            ax2ps.append(ax2 + 1.0)
            ay2ps.append(ay2 + 1.0)
            a1s.append(((ax2 - ax1) + 1.0) * ((ay2 - ay1) + 1.0))
            bnums.append(jnp.zeros((_LANES,), jnp.float32))
            bdens.append(jnp.ones((_LANES,), jnp.float32))
            bidxs.append(jnp.zeros((_LANES,), jnp.int32))
        for t in range(T):
            x1t = boxsv[0, t]
            y1t = boxsv[1, t]
            x2pt = boxsv[2, t]
            y2pt = boxsv[3, t]
            at = boxsv[4, t]
            for u in range(UNROLL):
                ltx = jnp.maximum(ax1s[u], x1t)
                lty = jnp.maximum(ay1s[u], y1t)
                rxp = jnp.minimum(ax2ps[u], x2pt)
                ryp = jnp.minimum(ay2ps[u], y2pt)
                wx = jnp.maximum(rxp - ltx, 0.0)
                wy = jnp.maximum(ryp - lty, 0.0)
                inter = wx * wy
                s = a1s[u] + at
                better = inter * bdens[u] > bnums[u] * s
                bnums[u] = jnp.where(better, inter, bnums[u])
                bdens[u] = jnp.where(better, s, bdens[u])
                bidxs[u] = jnp.where(better, t, bidxs[u])
        for u in range(UNROLL):
            sl = sls[u]
            bidx = bidxs[u]
            miou = bnums[u] / (bdens[u] - bnums[u])
            lab = plsc.load_gather(boxgv, [bidx])
            bcx = plsc.load_gather(boxgv, [bidx + T])
            bcy = plsc.load_gather(boxgv, [bidx + 2 * T])
            bw = plsc.load_gather(boxgv, [bidx + 3 * T])
            bh = plsc.load_gather(boxgv, [bidx + 4 * T])
            ax = anchv[0, sl]
            ay = anchv[1, sl]
            aw = anchv[2, sl]
            ah = anchv[3, sl]
            ct = lab + 1.0
            ct = jnp.where(miou < 0.5, 0.0, ct)
            ct = jnp.where((miou > 0.4) & (miou < 0.5), -1.0, ct)
            outv[0, sl] = ct
            outv[1, sl] = (bcx - ax) / aw
            outv[2, sl] = (bcy - ay) / ah
            outv[3, sl] = bw / aw
            outv[4, sl] = bh / ah
        return carry

    lax.fori_loop(0, steps // UNROLL, step, 0)
    pltpu.sync_copy(outv, out_hbm.at[:, pl.ds(base, CH)])


def _fold_lanes(v, width):
    while width > 128 and width % 256 == 0:
        width //= 2
        v = v[:, :width] + v[:, width:]
    if width > 128:
        acc = v[:, 0:128]
        for k in range(1, width // 128):
            acc = acc + v[:, 128 * k:128 * (k + 1)]
        v = acc
    return v


def _f01(x):
    e = jnp.exp(-jnp.abs(x))
    s = jnp.maximum(x, 0.0) + jnp.log1p(e)
    t = x - s
    p2 = jnp.exp(t + t)
    q2 = jnp.exp(-(s + s))
    f0 = (1.0 - ALPHA) * p2 * s
    f1 = ALPHA * q2 * (-t)
    return f0, f1


def _fold20(f, C):
    t8 = f[0:8, :] + f[8:16, :]
    v4 = (t8[0:4, :] + t8[4:8, :]) + f[16:C, :]
    v2 = v4[0:2, :] + v4[2:4, :]
    return v2[0:1, :] + v2[1:2, :]


def _bg_body(C, A, BA, cls_ref, out_ref):
    lane = lax.broadcasted_iota(jnp.int32, (1, BA), 1)
    valid = (pl.program_id(0) * BA + lane) < A
    x = cls_ref[...]
    f0, _ = _f01(x)
    f0 = jnp.where(valid, f0, 0.0)
    out_ref[...] = _fold20(f0, C)


def _loss_body(C, A, BA, nsteps, cls_ref, loc_ref, match_ref, s0_ref,
               out_ref, acc):
    @pl.when(pl.program_id(0) == 0)
    def _init():
        acc[...] = jnp.zeros_like(acc)

    ct = match_ref[0:1, :]
    pos = ct > 0.0
    posf = pos.astype(jnp.float32)
    pn = (ct > -0.5).astype(jnp.float32)

    def _sl1(lp, enc):
        d = lp - jnp.where(pos, enc, 0.0)
        ad = jnp.abs(d)
        return jnp.where(ad < 1.0, 0.5 * d * d, ad - 0.5)

    sl1 = _sl1(loc_ref[0:2, :], match_ref[1:3, :]) + _sl1(
        loc_ref[2:4, :], jnp.log(match_ref[3:5, :]))
    loc_v = jnp.where(pos, sl1, 0.0)

    x = cls_ref[...]
    cti = ct.astype(jnp.int32) - 1
    cls_id = lax.broadcasted_iota(jnp.int32, x.shape, 0)
    xsel = _fold20(jnp.where(cls_id == cti, x, 0.0), C)
    g0, g1 = _f01(xsel)
    cls_v = pn * s0_ref[...] + jnp.where(pos, g1 - g0, 0.0)

    acc[0:1, :] += _fold_lanes(cls_v, cls_v.shape[1])
    acc[4:6, :] += _fold_lanes(loc_v, loc_v.shape[1])
    acc[8:9, :] += _fold_lanes(posf, posf.shape[1])

    @pl.when(pl.program_id(0) == nsteps - 1)
    def _fin():
        loc_s = jnp.sum(acc[4:6, :])
        cls_s = jnp.sum(acc[0:1, :])
        np_s = jnp.maximum(1.0, jnp.sum(acc[8:9, :]))
        out_ref[0] = (loc_s + cls_s) / np_s
        out_ref[1] = loc_s / np_s
        out_ref[2] = cls_s / np_s


def kernel(loc_preds, cls_preds, targets, iou_boxes):
    A = iou_boxes.shape[0]
    T = targets.shape[0]
    C = cls_preds.shape[-1]
    grid_n = -(-A // _BA)
    a_pad = -(-grid_n * _BA // (_NW * _LANES)) * _NW * _LANES
    grid_n = a_pad // _BA
    CH = a_pad // _NW
    steps = CH // _LANES

    boxes = targets[:, 2:] * IMG_SIZE
    labels = targets[:, 1]
    half = boxes[:, 2:] * 0.5
    xy1 = boxes[:, :2] - half
    xy2 = boxes[:, :2] + half
    area = ((xy2[:, 0] - xy1[:, 0]) + 1.0) * ((xy2[:, 1] - xy1[:, 1]) + 1.0)
    box_scan = jnp.stack(
        [xy1[:, 0], xy1[:, 1], xy2[:, 0] + 1.0, xy2[:, 1] + 1.0, area], axis=0)
    box_scan = jnp.broadcast_to(box_scan[:, :, None], (5, T, _LANES))
    box_gath = jnp.concatenate(
        [labels, boxes[:, 0], boxes[:, 1], boxes[:, 2], boxes[:, 3]], axis=0)

    npad = a_pad - A
    anch_pad = jnp.tile(
        jnp.array([[-1e6], [-1e6], [1.0], [1.0]], jnp.float32), (1, npad))
    anchT = jnp.concatenate([iou_boxes.T, anch_pad], axis=1)
    clsT = cls_preds[0].T
    locT = loc_preds[0].T

    mesh = plsc.VectorSubcoreMesh(
        core_axis_name="c", subcore_axis_name="s",
        num_cores=_NC, num_subcores=_NS)
    match = pl.kernel(
        functools.partial(_match_body, T, CH, steps),
        out_type=jax.ShapeDtypeStruct((5, a_pad), jnp.float32),
        mesh=mesh,
        scratch_types=[
            pltpu.VMEM((5, T, _LANES), jnp.float32),
            pltpu.VMEM((5 * T,), jnp.float32),
            pltpu.VMEM((4, CH), jnp.float32),
            pltpu.VMEM((5, CH), jnp.float32),
        ],
        compiler_params=pltpu.CompilerParams(needs_layout_passes=False),
    )(box_scan, box_gath, anchT)

    s0 = pl.pallas_call(
        functools.partial(_bg_body, C, A, _BA),
        grid=(grid_n,),
        in_specs=[pl.BlockSpec((C, _BA), lambda i: (0, i))],
        out_specs=pl.BlockSpec((1, _BA), lambda i: (0, i)),
        out_shape=jax.ShapeDtypeStruct((1, a_pad), jnp.float32),
    )(clsT)

    sums = pl.pallas_call(
        functools.partial(_loss_body, C, A, _BA, grid_n),
        grid=(grid_n,),
        in_specs=[
            pl.BlockSpec((C, _BA), lambda i: (0, i)),
            pl.BlockSpec((4, _BA), lambda i: (0, i)),
            pl.BlockSpec((5, _BA), lambda i: (0, i)),
            pl.BlockSpec((1, _BA), lambda i: (0, i)),
        ],
        out_specs=pl.BlockSpec(memory_space=pltpu.SMEM),
        out_shape=jax.ShapeDtypeStruct((3,), jnp.float32),
        scratch_shapes=[pltpu.VMEM((16, 128), jnp.float32)],
    )(clsT, locT, match, s0)

    return (sums[0], sums[1], sums[2])
